# Initial kernel scaffold; baseline (speedup 1.0000x reference)
import jax
import jax.numpy as jnp
from jax.experimental import pallas as pl


def kernel(x, edge_index, W1, W2):
    raise NotImplementedError("write your pallas kernel here")


# speedup vs baseline: 6.5854x; 6.5854x over previous
import functools

import jax
import jax.numpy as jnp
from jax import lax
from jax.experimental import pallas as pl
from jax.experimental.pallas import tpu as pltpu
from jax.experimental.pallas import tpu_sc as plsc

N_NODES = 10000
N_EDGES = 160000
D_IN = 256
D_HID = 32
D_OUT = 256

NPAD = 10240
EPAD = 163840
N_TILES = 32
EDGES_PER_TILE = EPAD // N_TILES
CHUNK = 128
CHUNKS_PER_TILE = EDGES_PER_TILE // CHUNK
SHARD = NPAD // 16
ROW_BLK = 1024
GRID = NPAD // ROW_BLK


def _mm1_body(x_ref, w_ref, o_ref):
    o_ref[...] = lax.dot_general(
        x_ref[...], w_ref[...], (((1,), (1,)), ((), ())),
        preferred_element_type=jnp.float32)


def _combine_relu_body(agg_ref, y_ref, o_ref):
    o_ref[...] = jnp.maximum(agg_ref[0] + agg_ref[1] + y_ref[...], 0.0)


def _mm2_body(agg_ref, h_ref, w_ref, o_ref):
    t = agg_ref[0] + agg_ref[1] + h_ref[...]
    o_ref[...] = lax.dot_general(
        t, w_ref[...], (((1,), (1,)), ((), ())),
        preferred_element_type=jnp.float32)


def _mm1(x_pad, w1):
    return pl.pallas_call(
        _mm1_body,
        grid=(GRID,),
        in_specs=[
            pl.BlockSpec((ROW_BLK, D_IN), lambda i: (i, 0)),
            pl.BlockSpec((D_HID, D_IN), lambda i: (0, 0)),
        ],
        out_specs=pl.BlockSpec((ROW_BLK, D_HID), lambda i: (i, 0)),
        out_shape=jax.ShapeDtypeStruct((NPAD, D_HID), jnp.float32),
    )(x_pad, w1)


def _combine_relu(agg, y):
    return pl.pallas_call(
        _combine_relu_body,
        grid=(GRID,),
        in_specs=[
            pl.BlockSpec((2, ROW_BLK, D_HID), lambda i: (0, i, 0)),
            pl.BlockSpec((ROW_BLK, D_HID), lambda i: (i, 0)),
        ],
        out_specs=pl.BlockSpec((ROW_BLK, D_HID), lambda i: (i, 0)),
        out_shape=jax.ShapeDtypeStruct((NPAD, D_HID), jnp.float32),
    )(agg, y)


def _mm2(agg, h, w2):
    return pl.pallas_call(
        _mm2_body,
        grid=(GRID,),
        in_specs=[
            pl.BlockSpec((2, ROW_BLK, D_HID), lambda i: (0, i, 0)),
            pl.BlockSpec((ROW_BLK, D_HID), lambda i: (i, 0)),
            pl.BlockSpec((D_OUT, D_HID), lambda i: (0, 0)),
        ],
        out_specs=pl.BlockSpec((ROW_BLK, D_OUT), lambda i: (i, 0)),
        out_shape=jax.ShapeDtypeStruct((NPAD, D_OUT), jnp.float32),
    )(agg, h, w2)


def _sc_scatter_body(y_hbm, src_hbm, dst_hbm, out_hbm,
                     sidx, didx, rows, stage, agg, sem):
    c = lax.axis_index("c")
    s = lax.axis_index("s")
    wid = c * 16 + s

    def _zero_row(i, carry):
        z = jnp.zeros((16,), jnp.float32)
        stage[i, 0:16] = z
        stage[i, 16:32] = z
        return carry
    lax.fori_loop(0, SHARD, _zero_row, 0)
    pltpu.sync_copy(stage, agg.at[pl.ds(s * SHARD, SHARD)])
    plsc.subcore_barrier()

    base0 = wid * EDGES_PER_TILE

    def _chunk(i, carry):
        b = base0 + i * CHUNK
        pltpu.sync_copy(src_hbm.at[pl.ds(b, CHUNK)], sidx)
        pltpu.sync_copy(dst_hbm.at[pl.ds(b, CHUNK)], didx)
        pltpu.async_copy(y_hbm.at[sidx], rows, sem).wait()
        pltpu.sync_copy(rows, agg.at[didx], add=True)
        return carry
    lax.fori_loop(0, CHUNKS_PER_TILE, _chunk, 0)
    plsc.subcore_barrier()

    pltpu.sync_copy(agg.at[pl.ds(s * SHARD, SHARD)], stage)
    pltpu.sync_copy(stage, out_hbm.at[c, pl.ds(s * SHARD, SHARD)])


_sc_scatter = pl.kernel(
    _sc_scatter_body,
    out_type=jax.ShapeDtypeStruct((2, NPAD, D_HID), jnp.float32),
    mesh=plsc.VectorSubcoreMesh(core_axis_name="c", subcore_axis_name="s"),
    scratch_types=[
        pltpu.VMEM((CHUNK,), jnp.int32),
        pltpu.VMEM((CHUNK,), jnp.int32),
        pltpu.VMEM((CHUNK, D_HID), jnp.float32),
        pltpu.VMEM((SHARD, D_HID), jnp.float32),
        pltpu.VMEM_SHARED((NPAD, D_HID), jnp.float32),
        pltpu.SemaphoreType.DMA,
    ],
    compiler_params=pltpu.CompilerParams(use_tc_tiling_on_sc=False),
)


@jax.jit
def kernel(x, edge_index, W1, W2):
    src = edge_index[0].astype(jnp.int32)
    dst = edge_index[1].astype(jnp.int32)
    src_pad = jnp.full((EPAD,), N_NODES, jnp.int32).at[:N_EDGES].set(src)
    dst_pad = jnp.full((EPAD,), N_NODES, jnp.int32).at[:N_EDGES].set(dst)
    x_pad = jnp.zeros((NPAD, D_IN), jnp.float32).at[:N_NODES].set(x)

    y = _mm1(x_pad, W1)
    agg1 = _sc_scatter(y, src_pad, dst_pad)
    h = _combine_relu(agg1, y)
    agg2 = _sc_scatter(h, src_pad, dst_pad)
    out = _mm2(agg2, h, W2)
    return out[:N_NODES]


# speedup vs baseline: 9.1109x; 1.3835x over previous
import jax
import jax.numpy as jnp
from jax import lax
from jax.experimental import pallas as pl
from jax.experimental.pallas import tpu as pltpu
from jax.experimental.pallas import tpu_sc as plsc

N_NODES = 10000
N_EDGES = 160000
D_IN = 256
D_HID = 32
D_OUT = 256

NACC = 10016
EPAD = 163840
N_TILES = 32
CHUNK = 128
CHUNKS_PER_TILE = EPAD // (N_TILES * CHUNK)
NBUF = 4
GROUPS = CHUNKS_PER_TILE // NBUF
SHARD = NACC // 16
ROW_BLK = 1000
GRID = N_NODES // ROW_BLK


def _mm1_body(x_ref, w_ref, o_ref):
    o_ref[...] = lax.dot_general(
        x_ref[...], w_ref[...], (((1,), (1,)), ((), ())),
        preferred_element_type=jnp.float32)


def _combine_relu_body(agg_ref, y_ref, o_ref):
    o_ref[...] = jnp.maximum(agg_ref[0] + agg_ref[1] + y_ref[...], 0.0)


def _mm2_body(agg_ref, h_ref, w_ref, o_ref):
    t = agg_ref[0] + agg_ref[1] + h_ref[...]
    o_ref[...] = lax.dot_general(
        t, w_ref[...], (((1,), (1,)), ((), ())),
        preferred_element_type=jnp.float32)


def _mm1(x, w1):
    return pl.pallas_call(
        _mm1_body,
        grid=(GRID,),
        in_specs=[
            pl.BlockSpec((ROW_BLK, D_IN), lambda i: (i, 0)),
            pl.BlockSpec((D_HID, D_IN), lambda i: (0, 0)),
        ],
        out_specs=pl.BlockSpec((ROW_BLK, D_HID), lambda i: (i, 0)),
        out_shape=jax.ShapeDtypeStruct((N_NODES, D_HID), jnp.float32),
    )(x, w1)


def _combine_relu(agg, y):
    return pl.pallas_call(
        _combine_relu_body,
        grid=(GRID,),
        in_specs=[
            pl.BlockSpec((2, ROW_BLK, D_HID), lambda i: (0, i, 0)),
            pl.BlockSpec((ROW_BLK, D_HID), lambda i: (i, 0)),
        ],
        out_specs=pl.BlockSpec((ROW_BLK, D_HID), lambda i: (i, 0)),
        out_shape=jax.ShapeDtypeStruct((N_NODES, D_HID), jnp.float32),
    )(agg, y)


def _mm2(agg, h, w2):
    return pl.pallas_call(
        _mm2_body,
        grid=(GRID,),
        in_specs=[
            pl.BlockSpec((2, ROW_BLK, D_HID), lambda i: (0, i, 0)),
            pl.BlockSpec((ROW_BLK, D_HID), lambda i: (i, 0)),
            pl.BlockSpec((D_OUT, D_HID), lambda i: (0, 0)),
        ],
        out_specs=pl.BlockSpec((ROW_BLK, D_OUT), lambda i: (i, 0)),
        out_shape=jax.ShapeDtypeStruct((N_NODES, D_OUT), jnp.float32),
    )(agg, h, w2)


def _sc_scatter_body(y_hbm, src_hbm, dst_hbm, out_hbm,
                     sidx, didx, rows0, rows1, rows2, rows3, stage, agg,
                     sem0, sem1, sem2, sem3):
    c = lax.axis_index("c")
    s = lax.axis_index("s")
    wid = c * 16 + s
    rows = [rows0, rows1, rows2, rows3]
    sems = [sem0, sem1, sem2, sem3]

    pltpu.sync_copy(src_hbm.at[pl.ds(wid * CHUNKS_PER_TILE, CHUNKS_PER_TILE)],
                    sidx)
    pltpu.sync_copy(dst_hbm.at[pl.ds(wid * CHUNKS_PER_TILE, CHUNKS_PER_TILE)],
                    didx)

    def _zero_row(i, carry):
        z = jnp.zeros((16,), jnp.float32)
        stage[i, 0:16] = z
        stage[i, 16:32] = z
        return carry
    lax.fori_loop(0, SHARD, _zero_row, 0)
    pltpu.sync_copy(stage, agg.at[pl.ds(s * SHARD, SHARD)])
    plsc.subcore_barrier()

    def _group(g, carry):
        handles = []
        for b in range(NBUF):
            j = g * NBUF + b
            handles.append(
                pltpu.async_copy(y_hbm.at[sidx.at[j]], rows[b], sems[b]))
        for b in range(NBUF):
            j = g * NBUF + b
            handles[b].wait()
            pltpu.sync_copy(rows[b], agg.at[didx.at[j]], add=True)
        return carry
    lax.fori_loop(0, GROUPS, _group, 0)
    plsc.subcore_barrier()

    pltpu.sync_copy(agg.at[pl.ds(s * SHARD, SHARD)], stage)
    pltpu.sync_copy(stage, out_hbm.at[c, pl.ds(s * SHARD, SHARD)])


_sc_scatter = pl.kernel(
    _sc_scatter_body,
    out_type=jax.ShapeDtypeStruct((2, NACC, D_HID), jnp.float32),
    mesh=plsc.VectorSubcoreMesh(core_axis_name="c", subcore_axis_name="s"),
    scratch_types=[
        pltpu.VMEM((CHUNKS_PER_TILE, CHUNK), jnp.int32),
        pltpu.VMEM((CHUNKS_PER_TILE, CHUNK), jnp.int32),
        pltpu.VMEM((CHUNK, D_HID), jnp.float32),
        pltpu.VMEM((CHUNK, D_HID), jnp.float32),
        pltpu.VMEM((CHUNK, D_HID), jnp.float32),
        pltpu.VMEM((CHUNK, D_HID), jnp.float32),
        pltpu.VMEM((SHARD, D_HID), jnp.float32),
        pltpu.VMEM_SHARED((NACC, D_HID), jnp.float32),
        pltpu.SemaphoreType.DMA,
        pltpu.SemaphoreType.DMA,
        pltpu.SemaphoreType.DMA,
        pltpu.SemaphoreType.DMA,
    ],
    compiler_params=pltpu.CompilerParams(use_tc_tiling_on_sc=False),
)


@jax.jit
def kernel(x, edge_index, W1, W2):
    src = edge_index[0].astype(jnp.int32)
    dst = edge_index[1].astype(jnp.int32)
    src_pad = jnp.zeros((EPAD,), jnp.int32).at[:N_EDGES].set(src)
    dst_pad = jnp.full((EPAD,), N_NODES, jnp.int32).at[:N_EDGES].set(dst)
    src2d = src_pad.reshape(EPAD // CHUNK, CHUNK)
    dst2d = dst_pad.reshape(EPAD // CHUNK, CHUNK)

    y = _mm1(x, W1)
    agg1 = _sc_scatter(y, src2d, dst2d)
    h = _combine_relu(agg1, y)
    agg2 = _sc_scatter(h, src2d, dst2d)
    return _mm2(agg2, h, W2)


# speedup vs baseline: 14.2094x; 1.5596x over previous
import jax
import jax.numpy as jnp
from jax import lax
from jax.experimental import pallas as pl
from jax.experimental.pallas import tpu as pltpu
from jax.experimental.pallas import tpu_sc as plsc

N_NODES = 10000
N_EDGES = 160000
D_IN = 256
D_HID = 32
D_OUT = 256

NACC = 10016
EPAD = 163840
N_TILES = 32
CHUNK = 128
CHUNKS_PER_TILE = EPAD // (N_TILES * CHUNK)
NBUF = 4
GROUPS = CHUNKS_PER_TILE // NBUF
SHARD = NACC // 16
ROW_BLK = 1000
GRID = N_NODES // ROW_BLK


def _mm1_body(x_ref, w_ref, o_ref):
    o_ref[...] = lax.dot_general(
        x_ref[...], w_ref[...], (((1,), (1,)), ((), ())),
        preferred_element_type=jnp.float32)


def _combine_relu_body(agg_ref, y_ref, o_ref):
    o_ref[...] = jnp.maximum(agg_ref[0] + agg_ref[1] + y_ref[...], 0.0)


def _mm2_body(agg_ref, h_ref, w_ref, o_ref):
    t = agg_ref[0] + agg_ref[1] + h_ref[...]
    o_ref[...] = lax.dot_general(
        t, w_ref[...], (((1,), (1,)), ((), ())),
        preferred_element_type=jnp.float32)


def _mm1(x, w1):
    return pl.pallas_call(
        _mm1_body,
        grid=(GRID,),
        in_specs=[
            pl.BlockSpec((ROW_BLK, D_IN), lambda i: (i, 0)),
            pl.BlockSpec((D_HID, D_IN), lambda i: (0, 0)),
        ],
        out_specs=pl.BlockSpec((ROW_BLK, D_HID), lambda i: (i, 0)),
        out_shape=jax.ShapeDtypeStruct((N_NODES, D_HID), jnp.float32),
    )(x, w1)


def _combine_relu(agg, y):
    return pl.pallas_call(
        _combine_relu_body,
        grid=(GRID,),
        in_specs=[
            pl.BlockSpec((2, ROW_BLK, D_HID), lambda i: (0, i, 0)),
            pl.BlockSpec((ROW_BLK, D_HID), lambda i: (i, 0)),
        ],
        out_specs=pl.BlockSpec((ROW_BLK, D_HID), lambda i: (i, 0)),
        out_shape=jax.ShapeDtypeStruct((N_NODES, D_HID), jnp.float32),
    )(agg, y)


def _mm2(agg, h, w2):
    return pl.pallas_call(
        _mm2_body,
        grid=(GRID,),
        in_specs=[
            pl.BlockSpec((2, ROW_BLK, D_HID), lambda i: (0, i, 0)),
            pl.BlockSpec((ROW_BLK, D_HID), lambda i: (i, 0)),
            pl.BlockSpec((D_OUT, D_HID), lambda i: (0, 0)),
        ],
        out_specs=pl.BlockSpec((ROW_BLK, D_OUT), lambda i: (i, 0)),
        out_shape=jax.ShapeDtypeStruct((N_NODES, D_OUT), jnp.float32),
    )(agg, h, w2)


def _sc_scatter_body(y_hbm, src_hbm, dst_hbm, out_hbm,
                     sidx, didx, rows0, rows1, rows2, rows3, stage, ysh, agg,
                     sem0, sem1, sem2, sem3):
    c = lax.axis_index("c")
    s = lax.axis_index("s")
    wid = c * 16 + s
    rows = [rows0, rows1, rows2, rows3]
    sems = [sem0, sem1, sem2, sem3]

    pltpu.sync_copy(src_hbm.at[pl.ds(wid * CHUNKS_PER_TILE, CHUNKS_PER_TILE)],
                    sidx)
    pltpu.sync_copy(dst_hbm.at[pl.ds(wid * CHUNKS_PER_TILE, CHUNKS_PER_TILE)],
                    didx)

    YSHARD = N_NODES // 16
    pltpu.sync_copy(y_hbm.at[pl.ds(s * YSHARD, YSHARD)],
                    stage.at[pl.ds(0, YSHARD)])
    pltpu.sync_copy(stage.at[pl.ds(0, YSHARD)],
                    ysh.at[pl.ds(s * YSHARD, YSHARD)])

    def _zero_row(i, carry):
        z = jnp.zeros((16,), jnp.float32)
        stage[i, 0:16] = z
        stage[i, 16:32] = z
        return carry
    lax.fori_loop(0, SHARD, _zero_row, 0)
    pltpu.sync_copy(stage, agg.at[pl.ds(s * SHARD, SHARD)])
    plsc.subcore_barrier()

    def _group(g, carry):
        handles = []
        for b in range(NBUF):
            j = g * NBUF + b
            handles.append(
                pltpu.async_copy(ysh.at[sidx.at[j]], rows[b], sems[b]))
        for b in range(NBUF):
            j = g * NBUF + b
            handles[b].wait()
            pltpu.sync_copy(rows[b], agg.at[didx.at[j]], add=True)
        return carry
    lax.fori_loop(0, GROUPS, _group, 0)
    plsc.subcore_barrier()

    pltpu.sync_copy(agg.at[pl.ds(s * SHARD, SHARD)], stage)
    pltpu.sync_copy(stage, out_hbm.at[c, pl.ds(s * SHARD, SHARD)])


_sc_scatter = pl.kernel(
    _sc_scatter_body,
    out_type=jax.ShapeDtypeStruct((2, NACC, D_HID), jnp.float32),
    mesh=plsc.VectorSubcoreMesh(core_axis_name="c", subcore_axis_name="s"),
    scratch_types=[
        pltpu.VMEM((CHUNKS_PER_TILE, CHUNK), jnp.int32),
        pltpu.VMEM((CHUNKS_PER_TILE, CHUNK), jnp.int32),
        pltpu.VMEM((CHUNK, D_HID), jnp.float32),
        pltpu.VMEM((CHUNK, D_HID), jnp.float32),
        pltpu.VMEM((CHUNK, D_HID), jnp.float32),
        pltpu.VMEM((CHUNK, D_HID), jnp.float32),
        pltpu.VMEM((SHARD, D_HID), jnp.float32),
        pltpu.VMEM_SHARED((N_NODES, D_HID), jnp.float32),
        pltpu.VMEM_SHARED((NACC, D_HID), jnp.float32),
        pltpu.SemaphoreType.DMA,
        pltpu.SemaphoreType.DMA,
        pltpu.SemaphoreType.DMA,
        pltpu.SemaphoreType.DMA,
    ],
    compiler_params=pltpu.CompilerParams(use_tc_tiling_on_sc=False),
)


@jax.jit
def kernel(x, edge_index, W1, W2):
    src = edge_index[0].astype(jnp.int32)
    dst = edge_index[1].astype(jnp.int32)
    src_pad = jnp.zeros((EPAD,), jnp.int32).at[:N_EDGES].set(src)
    dst_pad = jnp.full((EPAD,), N_NODES, jnp.int32).at[:N_EDGES].set(dst)
    src2d = src_pad.reshape(EPAD // CHUNK, CHUNK)
    dst2d = dst_pad.reshape(EPAD // CHUNK, CHUNK)

    y = _mm1(x, W1)
    agg1 = _sc_scatter(y, src2d, dst2d)
    h = _combine_relu(agg1, y)
    agg2 = _sc_scatter(h, src2d, dst2d)
    return _mm2(agg2, h, W2)


# speedup vs baseline: 19.6035x; 1.3796x over previous
import jax
import jax.numpy as jnp
from jax import lax
from jax.experimental import pallas as pl
from jax.experimental.pallas import tpu as pltpu
from jax.experimental.pallas import tpu_sc as plsc

N_NODES = 10000
N_EDGES = 160000
D_IN = 256
D_HID = 32
D_OUT = 256

CHUNK = 128
N_CHUNKS = N_EDGES // CHUNK
N_TILES = 32
BASE_CH = N_CHUNKS // N_TILES
EXTRA = N_CHUNKS - BASE_CH * N_TILES
NBUF = 4
GROUPS = BASE_CH // NBUF
SHARD = N_NODES // 16
ROW_BLK = 2000
GRID = N_NODES // ROW_BLK


def _mm1_body(x_ref, w_ref, o_ref):
    o_ref[...] = lax.dot_general(
        x_ref[...], w_ref[...], (((1,), (1,)), ((), ())),
        preferred_element_type=jnp.float32)


def _mm2_body(q_ref, w_ref, o_ref):
    t = q_ref[0] + q_ref[1]
    o_ref[...] = lax.dot_general(
        t, w_ref[...], (((1,), (1,)), ((), ())),
        preferred_element_type=jnp.float32)


def _mm1(x, w1):
    return pl.pallas_call(
        _mm1_body,
        grid=(GRID,),
        in_specs=[
            pl.BlockSpec((ROW_BLK, D_IN), lambda i: (i, 0)),
            pl.BlockSpec((D_HID, D_IN), lambda i: (0, 0)),
        ],
        out_specs=pl.BlockSpec((ROW_BLK, D_HID), lambda i: (i, 0)),
        out_shape=jax.ShapeDtypeStruct((N_NODES, D_HID), jnp.float32),
    )(x, w1)


def _mm2(q, w2):
    return pl.pallas_call(
        _mm2_body,
        grid=(GRID,),
        in_specs=[
            pl.BlockSpec((2, ROW_BLK, D_HID), lambda i: (0, i, 0)),
            pl.BlockSpec((D_OUT, D_HID), lambda i: (0, 0)),
        ],
        out_specs=pl.BlockSpec((ROW_BLK, D_OUT), lambda i: (i, 0)),
        out_shape=jax.ShapeDtypeStruct((N_NODES, D_OUT), jnp.float32),
    )(q, w2)


def _edge_phase(ei, sidx, didx, rows, sems, ysh, agg, wid):
    pltpu.sync_copy(ei.at[0, pl.ds(wid * BASE_CH, BASE_CH)],
                    sidx.at[pl.ds(0, BASE_CH)])
    pltpu.sync_copy(ei.at[1, pl.ds(wid * BASE_CH, BASE_CH)],
                    didx.at[pl.ds(0, BASE_CH)])

    @pl.when(wid < EXTRA)
    def _():
        base = N_TILES * BASE_CH + wid
        pltpu.sync_copy(ei.at[0, pl.ds(base, 1)], sidx.at[pl.ds(BASE_CH, 1)])
        pltpu.sync_copy(ei.at[1, pl.ds(base, 1)], didx.at[pl.ds(BASE_CH, 1)])

    plsc.subcore_barrier()

    def _group(g, carry):
        handles = []
        for b in range(NBUF):
            j = g * NBUF + b
            handles.append(
                pltpu.async_copy(ysh.at[sidx.at[j]], rows[b], sems[b]))
        for b in range(NBUF):
            j = g * NBUF + b
            handles[b].wait()
            pltpu.sync_copy(rows[b], agg.at[didx.at[j]], add=True)
        return carry
    lax.fori_loop(0, GROUPS, _group, 0)

    for j in range(GROUPS * NBUF, BASE_CH):
        b = j % NBUF
        pltpu.async_copy(ysh.at[sidx.at[j]], rows[b], sems[b]).wait()
        pltpu.sync_copy(rows[b], agg.at[didx.at[j]], add=True)

    @pl.when(wid < EXTRA)
    def _():
        pltpu.async_copy(ysh.at[sidx.at[BASE_CH]], rows[0], sems[0]).wait()
        pltpu.sync_copy(rows[0], agg.at[didx.at[BASE_CH]], add=True)

    plsc.subcore_barrier()


def _sc_layer1_body(y_hbm, ei, out_hbm,
                    sidx, didx, rows0, rows1, rows2, rows3, buf, half,
                    ysh, agg, sem0, sem1, sem2, sem3):
    c = lax.axis_index("c")
    s = lax.axis_index("s")
    wid = c * 16 + s
    rows = [rows0, rows1, rows2, rows3]
    sems = [sem0, sem1, sem2, sem3]

    pltpu.sync_copy(y_hbm.at[pl.ds(s * SHARD, SHARD)], buf)
    pltpu.sync_copy(buf, ysh.at[pl.ds(s * SHARD, SHARD)])

    def _halve(i, carry):
        half[i, 0:16] = buf[i, 0:16] * 0.5
        half[i, 16:32] = buf[i, 16:32] * 0.5
        return carry
    lax.fori_loop(0, SHARD, _halve, 0)
    pltpu.sync_copy(half, agg.at[pl.ds(s * SHARD, SHARD)])

    _edge_phase(ei, sidx, didx, rows, sems, ysh, agg, wid)

    pltpu.sync_copy(agg.at[pl.ds(s * SHARD, SHARD)], buf)
    pltpu.sync_copy(buf, out_hbm.at[c, pl.ds(s * SHARD, SHARD)])


def _sc_layer2_body(p_hbm, ei, out_hbm,
                    sidx, didx, rows0, rows1, rows2, rows3, bufa, bufb, half,
                    ysh, agg, sem0, sem1, sem2, sem3):
    c = lax.axis_index("c")
    s = lax.axis_index("s")
    wid = c * 16 + s
    rows = [rows0, rows1, rows2, rows3]
    sems = [sem0, sem1, sem2, sem3]

    pltpu.sync_copy(p_hbm.at[0, pl.ds(s * SHARD, SHARD)], bufa)
    pltpu.sync_copy(p_hbm.at[1, pl.ds(s * SHARD, SHARD)], bufb)

    def _mk_h(i, carry):
        h0 = jnp.maximum(bufa[i, 0:16] + bufb[i, 0:16], 0.0)
        h1 = jnp.maximum(bufa[i, 16:32] + bufb[i, 16:32], 0.0)
        bufa[i, 0:16] = h0
        bufa[i, 16:32] = h1
        half[i, 0:16] = h0 * 0.5
        half[i, 16:32] = h1 * 0.5
        return carry
    lax.fori_loop(0, SHARD, _mk_h, 0)
    pltpu.sync_copy(bufa, ysh.at[pl.ds(s * SHARD, SHARD)])
    pltpu.sync_copy(half, agg.at[pl.ds(s * SHARD, SHARD)])

    _edge_phase(ei, sidx, didx, rows, sems, ysh, agg, wid)

    pltpu.sync_copy(agg.at[pl.ds(s * SHARD, SHARD)], bufa)
    pltpu.sync_copy(bufa, out_hbm.at[c, pl.ds(s * SHARD, SHARD)])


_SC_MESH = plsc.VectorSubcoreMesh(core_axis_name="c", subcore_axis_name="s")
_SC_OUT = jax.ShapeDtypeStruct((2, N_NODES, D_HID), jnp.float32)
_COMMON_SCRATCH = [
    pltpu.VMEM((BASE_CH + 1, CHUNK), jnp.int32),
    pltpu.VMEM((BASE_CH + 1, CHUNK), jnp.int32),
    pltpu.VMEM((CHUNK, D_HID), jnp.float32),
    pltpu.VMEM((CHUNK, D_HID), jnp.float32),
    pltpu.VMEM((CHUNK, D_HID), jnp.float32),
    pltpu.VMEM((CHUNK, D_HID), jnp.float32),
]
_SHARED_SCRATCH = [
    pltpu.VMEM_SHARED((N_NODES, D_HID), jnp.float32),
    pltpu.VMEM_SHARED((N_NODES, D_HID), jnp.float32),
]
_SEMS = [pltpu.SemaphoreType.DMA] * 4

_sc_layer1 = pl.kernel(
    _sc_layer1_body,
    out_type=_SC_OUT,
    mesh=_SC_MESH,
    scratch_types=_COMMON_SCRATCH + [
        pltpu.VMEM((SHARD, D_HID), jnp.float32),
        pltpu.VMEM((SHARD, D_HID), jnp.float32),
    ] + _SHARED_SCRATCH + _SEMS,
    compiler_params=pltpu.CompilerParams(use_tc_tiling_on_sc=False),
)

_sc_layer2 = pl.kernel(
    _sc_layer2_body,
    out_type=_SC_OUT,
    mesh=_SC_MESH,
    scratch_types=_COMMON_SCRATCH + [
        pltpu.VMEM((SHARD, D_HID), jnp.float32),
        pltpu.VMEM((SHARD, D_HID), jnp.float32),
        pltpu.VMEM((SHARD, D_HID), jnp.float32),
    ] + _SHARED_SCRATCH + _SEMS,
    compiler_params=pltpu.CompilerParams(use_tc_tiling_on_sc=False),
)


@jax.jit
def kernel(x, edge_index, W1, W2):
    ei = edge_index.astype(jnp.int32).reshape(2, N_CHUNKS, CHUNK)
    y = _mm1(x, W1)
    p = _sc_layer1(y, ei)
    q = _sc_layer2(p, ei)
    return _mm2(q, W2)
